# Initial kernel scaffold; baseline (speedup 1.0000x reference)
import jax
import jax.numpy as jnp
from jax.experimental import pallas as pl


def kernel(loc_data, conf_data, dummy_a, dummy_b, priors, targets):
    raise NotImplementedError("write your pallas kernel here")


# speedup vs baseline: 34.7970x; 34.7970x over previous
import functools

import jax
import jax.numpy as jnp
from jax import lax
from jax.experimental import pallas as pl
from jax.experimental.pallas import tpu as pltpu

_NUM_CLASSES = 21
_NEGPOS_RATIO = 3
_VAR0, _VAR1 = 0.1, 0.2
_OVERLAP = 0.5
_P = 8732
_RS, _LS = 72, 128
_PP = _RS * _LS
_O = 16
_B = 32
_NEG_INF = float("-inf")


def _smooth_l1(d):
    ad = jnp.abs(d)
    return jnp.where(ad < 1.0, 0.5 * d * d, ad - 0.5)


def _key_of(bits):
    return jnp.where(bits >= 0, bits, bits ^ jnp.int32(0x7FFFFFFF))


def _mbl_body(tgt_ref, conf_ref, loc_ref, pri_ref, out_l, out_c, out_n,
              key_scr, k_scr):
    b = pl.program_id(0)

    ridx = lax.broadcasted_iota(jnp.int32, (_RS, _LS), 0)
    lidx = lax.broadcasted_iota(jnp.int32, (_RS, _LS), 1)
    pidx = ridx * _LS + lidx
    valid = pidx < _P

    cx = pri_ref[0]
    cy = pri_ref[1]
    w = pri_ref[2]
    h = pri_ref[3]
    px0 = cx - w * 0.5
    py0 = cy - h * 0.5
    px1 = cx + w * 0.5
    py1 = cy + h * 0.5
    areas_b = (px1 - px0) * (py1 - py0)

    tx0 = [tgt_ref[0, t, 0] for t in range(_O)]
    ty0 = [tgt_ref[0, t, 1] for t in range(_O)]
    tx1 = [tgt_ref[0, t, 2] for t in range(_O)]
    ty1 = [tgt_ref[0, t, 3] for t in range(_O)]
    tlab = [tgt_ref[0, t, 4] for t in range(_O)]

    bto = jnp.zeros((_RS, _LS), jnp.float32)
    bti = jnp.zeros((_RS, _LS), jnp.int32)
    ovl = []
    for t in range(_O):
        iw = jnp.clip(jnp.minimum(tx1[t], px1) - jnp.maximum(tx0[t], px0), 0.0, None)
        ih = jnp.clip(jnp.minimum(ty1[t], py1) - jnp.maximum(ty0[t], py0), 0.0, None)
        inter = iw * ih
        area_a = (tx1[t] - tx0[t]) * (ty1[t] - ty0[t])
        ov = inter / (area_a + areas_b - inter)
        upd = ov > bto
        bti = jnp.where(upd, t, bti)
        bto = jnp.where(upd, ov, bto)
        ovl.append(ov)

    ovs = jnp.stack(ovl)
    m16 = jnp.max(jnp.max(ovs, axis=2), axis=1)
    cand = jnp.where(ovs == m16[:, None, None], pidx[None], _PP)
    bpi16 = jnp.min(jnp.min(cand, axis=2), axis=1)

    tno = lax.broadcasted_iota(jnp.int32, (_O, _RS, _LS), 0)
    win = jnp.max(jnp.where(pidx[None] == bpi16[:, None, None], tno, -1),
                  axis=0)
    forced = win >= 0
    bto = jnp.where(forced, 2.0, bto)
    bti = jnp.where(forced, win, bti)

    mx0 = jnp.zeros((_RS, _LS), jnp.float32)
    my0 = jnp.zeros((_RS, _LS), jnp.float32)
    mx1 = jnp.zeros((_RS, _LS), jnp.float32)
    my1 = jnp.zeros((_RS, _LS), jnp.float32)
    mlab = jnp.zeros((_RS, _LS), jnp.float32)
    for t in range(_O):
        sel = bti == t
        mx0 = jnp.where(sel, tx0[t], mx0)
        my0 = jnp.where(sel, ty0[t], my0)
        mx1 = jnp.where(sel, tx1[t], mx1)
        my1 = jnp.where(sel, ty1[t], my1)
        mlab = jnp.where(sel, tlab[t], mlab)

    conf_t = jnp.where(bto < _OVERLAP, 0, mlab.astype(jnp.int32))
    pos = conf_t > 0
    posf = pos.astype(jnp.float32)
    npos = jnp.sum(conf_t > 0, dtype=jnp.int32)

    g_cx = ((mx0 + mx1) * 0.5 - cx) / (_VAR0 * w)
    g_cy = ((my0 + my1) * 0.5 - cy) / (_VAR0 * h)
    g_w = jnp.log((mx1 - mx0) / w) * (1.0 / _VAR1)
    g_h = jnp.log((my1 - my0) / h) * (1.0 / _VAR1)
    sl = (_smooth_l1(loc_ref[0, 0] - g_cx) + _smooth_l1(loc_ref[0, 1] - g_cy)
          + _smooth_l1(loc_ref[0, 2] - g_w) + _smooth_l1(loc_ref[0, 3] - g_h))
    loss_l_img = jnp.sum(sl * posf)

    cls = [conf_ref[0, c] for c in range(_NUM_CLASSES)]
    rmax = functools.reduce(jnp.maximum, cls)
    ssum = jnp.zeros((_RS, _LS), jnp.float32)
    for c in range(_NUM_CLASSES):
        ssum = ssum + jnp.exp(cls[c] - rmax)
    lse = jnp.log(ssum) + rmax
    tgt = cls[0]
    for c in range(1, _NUM_CLASSES):
        tgt = jnp.where(conf_t == c, cls[c], tgt)
    ce = lse - tgt
    loss_c_pos = jnp.sum(ce * posf)

    v = jnp.where(valid, jnp.where(pos, 0.0, ce), _NEG_INF)
    k = jnp.minimum(_NEGPOS_RATIO * npos, _P - 1)

    key_scr[b] = _key_of(lax.bitcast_convert_type(v, jnp.int32))
    k_scr[b] = jnp.full((_LS,), k, jnp.int32)

    @pl.when(b == 0)
    def _():
        out_l[0, 0] = 0.0
        out_c[0, 0] = 0.0
        out_n[0, 0] = 0.0

    out_l[0, 0] += loss_l_img
    out_c[0, 0] += loss_c_pos
    out_n[0, 0] += npos.astype(jnp.float32)

    @pl.when(b == _B - 1)
    def _():
        keys = key_scr[...]
        kv = k_scr[...]

        def bs_body(_, lh):
            lo, hi = lh
            mid = (lo >> 1) + (hi >> 1) + (lo & hi & 1)
            cmp = (keys > mid[:, None, :]).astype(jnp.int32)
            cnt = jnp.sum(jnp.sum(cmp, axis=1), axis=1, keepdims=True)
            big = cnt >= kv
            live = lo < hi
            lo2 = jnp.where(live & big, mid + 1, lo)
            hi2 = jnp.where(live & (~big), mid, hi)
            return (lo2, hi2)

        lo0 = jnp.full((_B, _LS), -(2 ** 31), jnp.int32)
        hi0 = jnp.full((_B, _LS), 2 ** 31 - 1, jnp.int32)
        kth, _ = lax.fori_loop(0, 32, bs_body, (lo0, hi0))

        vall = lax.bitcast_convert_type(_key_of(keys), jnp.float32)
        gt = keys > kth[:, None, :]
        cnt_gt = jnp.sum(jnp.sum(gt.astype(jnp.int32), axis=1), axis=1,
                         keepdims=True)
        sum_gt = jnp.sum(jnp.sum(jnp.where(gt, vall, 0.0), axis=1), axis=1,
                         keepdims=True)
        tval = jnp.max(jnp.max(jnp.where(gt, _NEG_INF, vall), axis=1), axis=1,
                       keepdims=True)
        kcol = kv[:, 0:1]
        topk = sum_gt + (kcol - cnt_gt).astype(jnp.float32) * tval
        topk = jnp.where(kcol > 0, topk, 0.0)
        out_c[0, 0] += jnp.sum(topk)


def kernel(loc_data, conf_data, dummy_a, dummy_b, priors, targets):
    del dummy_a, dummy_b
    B = loc_data.shape[0]
    C = conf_data.shape[2]

    conf_p = jnp.transpose(conf_data, (0, 2, 1))
    conf_p = jnp.pad(conf_p, ((0, 0), (0, 0), (0, _PP - _P)))
    conf_p = conf_p.reshape(B, C, _RS, _LS)

    loc_p = jnp.transpose(loc_data, (0, 2, 1))
    loc_p = jnp.pad(loc_p, ((0, 0), (0, 0), (0, _PP - _P)))
    loc_p = loc_p.reshape(B, 4, _RS, _LS)

    fill = jnp.broadcast_to(jnp.array([10.0, 10.0, 1.0, 1.0], jnp.float32),
                            (_PP - _P, 4))
    pri_p = jnp.concatenate([priors[:_P], fill], axis=0)
    pri_p = jnp.transpose(pri_p).reshape(4, _RS, _LS)

    grid = (B,)
    out_l, out_c, out_n = pl.pallas_call(
        _mbl_body,
        grid=grid,
        in_specs=[
            pl.BlockSpec((1, _O, 5), lambda b: (b, 0, 0),
                         memory_space=pltpu.SMEM),
            pl.BlockSpec((1, C, _RS, _LS), lambda b: (b, 0, 0, 0)),
            pl.BlockSpec((1, 4, _RS, _LS), lambda b: (b, 0, 0, 0)),
            pl.BlockSpec((4, _RS, _LS), lambda b: (0, 0, 0)),
        ],
        out_specs=[
            pl.BlockSpec((1, 1), lambda b: (0, 0), memory_space=pltpu.SMEM),
            pl.BlockSpec((1, 1), lambda b: (0, 0), memory_space=pltpu.SMEM),
            pl.BlockSpec((1, 1), lambda b: (0, 0), memory_space=pltpu.SMEM),
        ],
        out_shape=[
            jax.ShapeDtypeStruct((1, 1), jnp.float32),
            jax.ShapeDtypeStruct((1, 1), jnp.float32),
            jax.ShapeDtypeStruct((1, 1), jnp.float32),
        ],
        scratch_shapes=[
            pltpu.VMEM((_B, _RS, _LS), jnp.int32),
            pltpu.VMEM((_B, _LS), jnp.int32),
        ],
        compiler_params=pltpu.CompilerParams(
            dimension_semantics=("arbitrary",)),
    )(targets, conf_p, loc_p, pri_p)

    total_pos = out_n[0, 0]
    N = jnp.where(total_pos > 0, total_pos, jnp.float32(B))
    return out_l[0, 0] / N, out_c[0, 0] / N
